# baseline (speedup 1.0000x reference)
import jax
import jax.numpy as jnp
from jax import lax
from jax.experimental import pallas as pl
from jax.experimental.pallas import tpu as pltpu
from jax.experimental.pallas import tpu_sc as plsc

_VOCAB = 100000
_HID = 128
_SEQ = 200
_BATCH = 1024
_EPS = 1e-12
_NW = 32
_TOK = _BATCH * _SEQ
_PER_W = _TOK // _NW
_CHUNK = 128
_NCHUNK = _PER_W // _CHUNK


def _rsqrt16(v):
    i = plsc.bitcast(v, jnp.int32)
    i = jnp.int32(0x5F3759DF) - (i >> 1)
    y = plsc.bitcast(i, jnp.float32)
    half = v * jnp.float32(0.5)
    for _ in range(3):
        y = y * (jnp.float32(1.5) - half * y * y)
    return y


def _tec_body(ids_hbm, tt_hbm, word_hbm, pos_hbm, type_hbm, gam_hbm, bet_hbm,
              out_hbm, pt_v, ty_v, wrows_v, idx_v, ttc_v, gam_v, bet_v, sem):
    wid = lax.axis_index("s") * 2 + lax.axis_index("c")
    iota = lax.iota(jnp.int32, 16)

    pltpu.sync_copy(pos_hbm.at[pl.ds(0, _SEQ)], pt_v.at[pl.ds(0, _SEQ)])
    pltpu.sync_copy(pos_hbm.at[pl.ds(0, _SEQ)], pt_v.at[pl.ds(_SEQ, _SEQ)])
    pltpu.sync_copy(type_hbm, ty_v)
    pltpu.sync_copy(gam_hbm, gam_v)
    pltpu.sync_copy(bet_hbm, bet_v)

    def add_type(r, _):
        t = r // _SEQ
        for j in range(_HID // 16):
            sl = pl.ds(j * 16, 16)
            pt_v[r, sl] = pt_v[r, sl] + ty_v[t, sl]
        return 0
    lax.fori_loop(0, 2 * _SEQ, add_type, 0)

    zero16 = jnp.zeros((16,), jnp.float32)

    def chunk_body(c, _):
        base = wid * _PER_W + c * _CHUNK
        pltpu.sync_copy(ids_hbm.at[pl.ds(base, _CHUNK)], idx_v)
        pltpu.sync_copy(tt_hbm.at[pl.ds(base, _CHUNK)], ttc_v)
        pltpu.async_copy(word_hbm.at[idx_v], wrows_v, sem).wait()

        for g in range(_CHUNK // 16):
            tok = jnp.int32(g * 16) + iota
            tvec = ttc_v[pl.ds(g * 16, 16)]
            lvec = lax.rem(base + g * 16 + iota, jnp.int32(_SEQ))
            ptrow = tvec * _SEQ + lvec

            def p1(d, carry):
                s, ss = carry
                dv = jnp.full((16,), d, jnp.int32)
                w = plsc.load_gather(wrows_v, [tok, dv])
                p = plsc.load_gather(pt_v, [ptrow, dv])
                x = w + p
                plsc.store_scatter(wrows_v, [tok, dv], x)
                return (s + x, ss + x * x)

            s, ss = lax.fori_loop(0, _HID, p1, (zero16, zero16))
            mean = s * jnp.float32(1.0 / _HID)
            var = ss * jnp.float32(1.0 / _HID) - mean * mean
            rstd = _rsqrt16(var + jnp.float32(_EPS))

            def p2(d, _unused):
                dv = jnp.full((16,), d, jnp.int32)
                x = plsc.load_gather(wrows_v, [tok, dv])
                gd = plsc.load_gather(gam_v, [dv])
                bd = plsc.load_gather(bet_v, [dv])
                y = (x - mean) * rstd * gd + bd
                plsc.store_scatter(wrows_v, [tok, dv], y)
                return 0

            lax.fori_loop(0, _HID, p2, 0)

        pltpu.sync_copy(wrows_v, out_hbm.at[pl.ds(base, _CHUNK)])
        return 0

    lax.fori_loop(0, _NCHUNK, chunk_body, 0)


def kernel(input_ids, token_type_ids, word_emb, pos_emb, type_emb, ln_gamma, ln_beta):
    ids = input_ids.reshape(-1).astype(jnp.int32)
    tt = token_type_ids.reshape(-1).astype(jnp.int32)
    mesh = plsc.VectorSubcoreMesh(core_axis_name="c", subcore_axis_name="s")
    k = pl.kernel(
        _tec_body,
        out_type=jax.ShapeDtypeStruct((_TOK, _HID), jnp.float32),
        mesh=mesh,
        scratch_types=[
            pltpu.VMEM((2 * _SEQ, _HID), jnp.float32),
            pltpu.VMEM((2, _HID), jnp.float32),
            pltpu.VMEM((_CHUNK, _HID), jnp.float32),
            pltpu.VMEM((_CHUNK,), jnp.int32),
            pltpu.VMEM((_CHUNK,), jnp.int32),
            pltpu.VMEM((_HID,), jnp.float32),
            pltpu.VMEM((_HID,), jnp.float32),
            pltpu.SemaphoreType.DMA,
        ],
        compiler_params=pltpu.CompilerParams(needs_layout_passes=False),
    )
    out = k(ids, tt, word_emb, pos_emb, type_emb, ln_gamma, ln_beta)
    return out.reshape(_BATCH, _SEQ, _HID)


# speedup vs baseline: 6.4085x; 6.4085x over previous
import jax
import jax.numpy as jnp
from jax import lax
from jax.experimental import pallas as pl
from jax.experimental.pallas import tpu as pltpu
from jax.experimental.pallas import tpu_sc as plsc

_VOCAB = 100000
_HID = 128
_NV = _HID // 16
_SEQ = 200
_BATCH = 1024
_EPS = 1e-12
_NW = 32
_TOK = _BATCH * _SEQ
_PER_W = _TOK // _NW
_CHUNK = 128
_NCHUNK = _PER_W // _CHUNK
_UNROLL = 16


def _rsqrt16(v):
    i = plsc.bitcast(v, jnp.int32)
    i = jnp.int32(0x5F3759DF) - (i >> 1)
    y = plsc.bitcast(i, jnp.float32)
    half = v * jnp.float32(0.5)
    for _ in range(3):
        y = y * (jnp.float32(1.5) - half * y * y)
    return y


def _tree_sum(vs):
    while len(vs) > 1:
        vs = [a + b for a, b in zip(vs[::2], vs[1::2])]
    return vs[0]


def _tec_body(ids_hbm, tt_hbm, word_hbm, pos_hbm, type_hbm, gam_hbm, bet_hbm,
              out_hbm, pt_v, ty_v, wrows_v, idsw_v, ttw_v, gam_v, bet_v,
              gsem, osem):
    wid = lax.axis_index("s") * 2 + lax.axis_index("c")

    pltpu.sync_copy(ids_hbm.at[pl.ds(wid * _PER_W, _PER_W)], idsw_v)
    pltpu.sync_copy(tt_hbm.at[pl.ds(wid * _PER_W, _PER_W)], ttw_v)
    pltpu.sync_copy(pos_hbm.at[pl.ds(0, _SEQ)], pt_v.at[pl.ds(0, _SEQ)])
    pltpu.sync_copy(pos_hbm.at[pl.ds(0, _SEQ)], pt_v.at[pl.ds(_SEQ, _SEQ)])
    pltpu.sync_copy(type_hbm, ty_v)
    pltpu.sync_copy(gam_hbm, gam_v)
    pltpu.sync_copy(bet_hbm, bet_v)

    def add_type(r, _):
        t = r // _SEQ
        for j in range(_NV):
            sl = pl.ds(j * 16, 16)
            pt_v[r, sl] = pt_v[r, sl] + ty_v[t, sl]
        return 0
    lax.fori_loop(0, 2 * _SEQ, add_type, 0)

    gvs = [gam_v[pl.ds(j * 16, 16)] for j in range(_NV)]
    bvs = [bet_v[pl.ds(j * 16, 16)] for j in range(_NV)]

    def idx_slice(c):
        return idsw_v.at[pl.ds(pl.multiple_of(c * _CHUNK, _CHUNK), _CHUNK)]

    pltpu.async_copy(word_hbm.at[idx_slice(0)], wrows_v.at[0], gsem.at[0])

    def chunk_body(c, _):
        p = c & 1
        np_ = 1 - p

        @pl.when(c + 1 < _NCHUNK)
        def _start_next():
            @pl.when(c >= 1)
            def _wait_out():
                pltpu.make_async_copy(
                    wrows_v.at[np_], out_hbm.at[pl.ds(0, _CHUNK)], osem.at[np_]
                ).wait()
            pltpu.async_copy(
                word_hbm.at[idx_slice(c + 1)], wrows_v.at[np_], gsem.at[np_])

        pltpu.make_async_copy(
            word_hbm.at[idx_slice(c)], wrows_v.at[p], gsem.at[p]).wait()

        base = wid * _PER_W + c * _CHUNK

        def tok_body(it, _unused):
            tvec = ttw_v[pl.ds(c * _CHUNK + it * _UNROLL, 16)]
            for u in range(_UNROLL):
                i = it * _UNROLL + u
                t = tvec[u]
                l = lax.rem(base + i, _SEQ)
                pr = t * _SEQ + l
                xs = [wrows_v[p, i, pl.ds(j * 16, 16)] + pt_v[pr, pl.ds(j * 16, 16)]
                      for j in range(_NV)]
                m_v = jnp.full((16,), jnp.sum(_tree_sum(xs)), jnp.float32) \
                    * jnp.float32(1.0 / _HID)
                s2_v = jnp.full((16,), jnp.sum(_tree_sum([x * x for x in xs])),
                                jnp.float32) * jnp.float32(1.0 / _HID)
                var_v = s2_v - m_v * m_v
                rstd = _rsqrt16(var_v + jnp.float32(_EPS))
                c2 = m_v * rstd
                for j in range(_NV):
                    wrows_v[p, i, pl.ds(j * 16, 16)] = \
                        (xs[j] * rstd - c2) * gvs[j] + bvs[j]
            return 0

        lax.fori_loop(0, _CHUNK // _UNROLL, tok_body, 0)

        pltpu.async_copy(wrows_v.at[p], out_hbm.at[pl.ds(base, _CHUNK)],
                         osem.at[p])
        return 0

    lax.fori_loop(0, _NCHUNK, chunk_body, 0)

    pltpu.make_async_copy(
        wrows_v.at[0], out_hbm.at[pl.ds(0, _CHUNK)], osem.at[0]).wait()
    pltpu.make_async_copy(
        wrows_v.at[1], out_hbm.at[pl.ds(0, _CHUNK)], osem.at[1]).wait()


def kernel(input_ids, token_type_ids, word_emb, pos_emb, type_emb, ln_gamma, ln_beta):
    ids = input_ids.reshape(-1).astype(jnp.int32)
    tt = token_type_ids.reshape(-1).astype(jnp.int32)
    mesh = plsc.VectorSubcoreMesh(core_axis_name="c", subcore_axis_name="s")
    k = pl.kernel(
        _tec_body,
        out_type=jax.ShapeDtypeStruct((_TOK, _HID), jnp.float32),
        mesh=mesh,
        scratch_types=[
            pltpu.VMEM((2 * _SEQ, _HID), jnp.float32),
            pltpu.VMEM((2, _HID), jnp.float32),
            pltpu.VMEM((2, _CHUNK, _HID), jnp.float32),
            pltpu.VMEM((_PER_W,), jnp.int32),
            pltpu.VMEM((_PER_W,), jnp.int32),
            pltpu.VMEM((_HID,), jnp.float32),
            pltpu.VMEM((_HID,), jnp.float32),
            pltpu.SemaphoreType.DMA((2,)),
            pltpu.SemaphoreType.DMA((2,)),
        ],
        compiler_params=pltpu.CompilerParams(needs_layout_passes=False),
    )
    out = k(ids, tt, word_emb, pos_emb, type_emb, ln_gamma, ln_beta)
    return out.reshape(_BATCH, _SEQ, _HID)


# speedup vs baseline: 12.4096x; 1.9364x over previous
import jax
import jax.numpy as jnp
from jax import lax
from jax.experimental import pallas as pl
from jax.experimental.pallas import tpu as pltpu
from jax.experimental.pallas import tpu_sc as plsc

_VOCAB = 100000
_HID = 128
_NV = _HID // 16
_SEQ = 200
_BATCH = 1024
_EPS = 1e-12
_NW = 32
_TOK = _BATCH * _SEQ
_PER_W = _TOK // _NW
_CHUNK = 128
_NCHUNK = _PER_W // _CHUNK


def _rsqrt16(v):
    i = plsc.bitcast(v, jnp.int32)
    i = jnp.int32(0x5F3759DF) - (i >> 1)
    y = plsc.bitcast(i, jnp.float32)
    half = v * jnp.float32(0.5)
    for _ in range(3):
        y = y * (jnp.float32(1.5) - half * y * y)
    return y


def _tec_body(ids_hbm, tt_hbm, word_hbm, pos_hbm, type_hbm, gam_hbm, bet_hbm,
              out_hbm, ptbl_hbm, ty_v, wrows_v, ptx_v, idsw_v, ttw_v,
              gam_v, bet_v, stats_v, sums_v, ptidx_v, gsem, psem, osem):
    wid = lax.axis_index("s") * 2 + lax.axis_index("c")
    iota = lax.iota(jnp.int32, 16)

    pltpu.sync_copy(ids_hbm.at[pl.ds(wid * _PER_W, _PER_W)], idsw_v)
    pltpu.sync_copy(tt_hbm.at[pl.ds(wid * _PER_W, _PER_W)], ttw_v)
    pltpu.sync_copy(type_hbm, ty_v)
    pltpu.sync_copy(gam_hbm, gam_v)
    pltpu.sync_copy(bet_hbm, bet_v)

    for t in range(2):
        pltpu.sync_copy(pos_hbm.at[pl.ds(0, _CHUNK)], wrows_v.at[0])
        pltpu.sync_copy(pos_hbm.at[pl.ds(_CHUNK, _SEQ - _CHUNK)],
                        wrows_v.at[1].at[pl.ds(0, _SEQ - _CHUNK)])

        def add_ty(r, _):
            b = r // _CHUNK
            rr = lax.rem(r, _CHUNK)
            for j in range(_NV):
                sl = pl.ds(j * 16, 16)
                wrows_v[b, rr, sl] = wrows_v[b, rr, sl] + ty_v[t, sl]
            return 0
        lax.fori_loop(0, _SEQ, add_ty, 0)

        tb = wid * 2 * _SEQ + t * _SEQ
        pltpu.sync_copy(wrows_v.at[0], ptbl_hbm.at[pl.ds(tb, _CHUNK)])
        pltpu.sync_copy(wrows_v.at[1].at[pl.ds(0, _SEQ - _CHUNK)],
                        ptbl_hbm.at[pl.ds(tb + _CHUNK, _SEQ - _CHUNK)])

    def idx_slice(c):
        return idsw_v.at[pl.ds(pl.multiple_of(c * _CHUNK, _CHUNK), _CHUNK)]

    def fill_ptidx(cc, buf):
        for g in range(_CHUNK // 16):
            off = cc * _CHUNK + g * 16
            tvec = ttw_v[pl.ds(off, 16)]
            lvec = lax.rem(wid * _PER_W + off + iota, jnp.int32(_SEQ))
            ptidx_v[buf, pl.ds(g * 16, 16)] = \
                wid * (2 * _SEQ) + tvec * _SEQ + lvec

    def start_gathers(cc, buf):
        pltpu.async_copy(word_hbm.at[idx_slice(cc)], wrows_v.at[buf],
                         gsem.at[buf])
        pltpu.async_copy(ptbl_hbm.at[ptidx_v.at[buf]], ptx_v.at[buf],
                         psem.at[buf])

    fill_ptidx(0, 0)
    start_gathers(0, 0)

    m15 = lax.eq(iota, jnp.int32(15))
    gvs = [gam_v[pl.ds(j * 16, 16)] for j in range(_NV)]
    bvs = [bet_v[pl.ds(j * 16, 16)] for j in range(_NV)]

    def chunk_body(c, _):
        p = c & 1
        np_ = 1 - p

        @pl.when(c + 1 < _NCHUNK)
        def _start_next():
            @pl.when(c >= 1)
            def _wait_out():
                pltpu.make_async_copy(
                    wrows_v.at[np_], out_hbm.at[pl.ds(0, _CHUNK)], osem.at[np_]
                ).wait()
            fill_ptidx(c + 1, np_)
            start_gathers(c + 1, np_)

        pltpu.make_async_copy(
            word_hbm.at[idx_slice(c)], wrows_v.at[p], gsem.at[p]).wait()
        pltpu.make_async_copy(
            ptbl_hbm.at[ptidx_v.at[p]], ptx_v.at[p], psem.at[p]).wait()

        def grp_body(g, _unused):

            @plsc.parallel_loop(0, 16, step=1, unroll=8)
            def rpass(u):
                i = g * 16 + u
                fu = jnp.full((16,), u, jnp.int32)
                rsum = jnp.zeros((16,), jnp.float32)
                rsq = jnp.zeros((16,), jnp.float32)
                for j in range(_NV):
                    x = wrows_v[p, i, pl.ds(j * 16, 16)] \
                        + ptx_v[p, i, pl.ds(j * 16, 16)]
                    wrows_v[p, i, pl.ds(j * 16, 16)] = x
                    rsum = rsum + x
                    rsq = rsq + x * x
                cs = plsc.cumsum(rsum)
                cq = plsc.cumsum(rsq)
                plsc.store_scatter(sums_v, [fu], cs, mask=m15)
                plsc.store_scatter(sums_v, [fu + 16], cq, mask=m15)

            s1 = sums_v[pl.ds(0, 16)]
            s2 = sums_v[pl.ds(16, 16)]
            m_v = s1 * jnp.float32(1.0 / _HID)
            var_v = s2 * jnp.float32(1.0 / _HID) - m_v * m_v
            c1 = _rsqrt16(var_v + jnp.float32(_EPS))
            c2 = m_v * c1
            stats_v[pl.ds(0, 16)] = c1
            stats_v[pl.ds(16, 16)] = c2

            @plsc.parallel_loop(0, 16, step=1, unroll=4)
            def norm(u):
                i = g * 16 + u
                fu = jnp.full((16,), u, jnp.int32)
                c1u = plsc.load_gather(stats_v, [fu])
                c2u = plsc.load_gather(stats_v, [fu + 16])
                for j in range(_NV):
                    x = wrows_v[p, i, pl.ds(j * 16, 16)]
                    wrows_v[p, i, pl.ds(j * 16, 16)] = \
                        (x * c1u - c2u) * gvs[j] + bvs[j]
            return 0

        lax.fori_loop(0, _CHUNK // 16, grp_body, 0)

        base = wid * _PER_W + c * _CHUNK
        pltpu.async_copy(wrows_v.at[p], out_hbm.at[pl.ds(base, _CHUNK)],
                         osem.at[p])
        return 0

    lax.fori_loop(0, _NCHUNK, chunk_body, 0)

    pltpu.make_async_copy(
        wrows_v.at[0], out_hbm.at[pl.ds(0, _CHUNK)], osem.at[0]).wait()
    pltpu.make_async_copy(
        wrows_v.at[1], out_hbm.at[pl.ds(0, _CHUNK)], osem.at[1]).wait()


def kernel(input_ids, token_type_ids, word_emb, pos_emb, type_emb, ln_gamma, ln_beta):
    ids = input_ids.reshape(-1).astype(jnp.int32)
    tt = token_type_ids.reshape(-1).astype(jnp.int32)
    mesh = plsc.VectorSubcoreMesh(core_axis_name="c", subcore_axis_name="s")
    k = pl.kernel(
        _tec_body,
        out_type=[
            jax.ShapeDtypeStruct((_TOK, _HID), jnp.float32),
            jax.ShapeDtypeStruct((_NW * 2 * _SEQ, _HID), jnp.float32),
        ],
        mesh=mesh,
        scratch_types=[
            pltpu.VMEM((2, _HID), jnp.float32),
            pltpu.VMEM((2, _CHUNK, _HID), jnp.float32),
            pltpu.VMEM((2, _CHUNK, _HID), jnp.float32),
            pltpu.VMEM((_PER_W,), jnp.int32),
            pltpu.VMEM((_PER_W,), jnp.int32),
            pltpu.VMEM((_HID,), jnp.float32),
            pltpu.VMEM((_HID,), jnp.float32),
            pltpu.VMEM((32,), jnp.float32),
            pltpu.VMEM((32,), jnp.float32),
            pltpu.VMEM((2, _CHUNK), jnp.int32),
            pltpu.SemaphoreType.DMA((2,)),
            pltpu.SemaphoreType.DMA((2,)),
            pltpu.SemaphoreType.DMA((2,)),
        ],
        compiler_params=pltpu.CompilerParams(needs_layout_passes=False),
    )
    out, _ = k(ids, tt, word_emb, pos_emb, type_emb, ln_gamma, ln_beta)
    return out.reshape(_BATCH, _SEQ, _HID)
